# speedup vs baseline: 1.0563x; 1.0563x over previous
import functools

import jax
import jax.numpy as jnp
from jax.experimental import pallas as pl
from jax.experimental.pallas import tpu as pltpu

N = 10000
E = 320000
HEADS = 8
DH = 32
NP = 10240


def _mm_body(x_ref, w_ref, o_ref):
    o_ref[...] = jnp.dot(x_ref[...], w_ref[...],
                         preferred_element_type=jnp.float32)


def _matmul(x, w):
    m, k = x.shape
    _, n = w.shape
    bm = 1024
    return pl.pallas_call(
        _mm_body,
        grid=(m // bm,),
        in_specs=[pl.BlockSpec((bm, k), lambda i: (i, 0)),
                  pl.BlockSpec((k, n), lambda i: (0, 0))],
        out_specs=pl.BlockSpec((bm, n), lambda i: (i, 0)),
        out_shape=jax.ShapeDtypeStruct((m, n), jnp.float32),
    )(x, w)


def _gat_layer(xp, src, dst, W, al, ar, b):
    h = _matmul(xp, W)[:N].reshape(N, HEADS, DH)
    el = jnp.sum(h * al[None], -1)
    er = jnp.sum(h * ar[None], -1)
    M = jnp.max(el, 0) + jnp.max(er, 0)
    t = el[src] + er[dst]
    e = jax.nn.leaky_relu(t, 0.2)
    ex = jnp.exp(e - M[None])
    den = jax.ops.segment_sum(ex, dst, num_segments=N)
    num = jax.ops.segment_sum(h[src] * ex[:, :, None], dst, num_segments=N)
    out = num / (den[:, :, None] + 1e-9)
    return out.reshape(N, HEADS * DH) + b[None]


def kernel(x, edge_index, W1, al1, ar1, b1, W2, al2, ar2, b2):
    src, dst = edge_index[0], edge_index[1]
    xp = jnp.pad(x, ((0, NP - N), (0, 0)))
    h1 = jax.nn.relu(_gat_layer(xp, src, dst, W1, al1, ar1, b1))
    h1p = jnp.pad(h1, ((0, NP - N), (0, 0)))
    return _gat_layer(h1p, src, dst, W2, al2, ar2, b2)


# speedup vs baseline: 19.6617x; 18.6144x over previous
import functools

import jax
import jax.numpy as jnp
from jax import lax
from jax.experimental import pallas as pl
from jax.experimental.pallas import tpu as pltpu
from jax.experimental.pallas import tpu_sc as plsc

N = 10000
E = 320000
HEADS = 8
DH = 32
HD = HEADS * DH
HALF = HD // 2
NP = 10240
BM = 1024
NSC = 2
NTILE = 16
NW = NSC * NTILE
EPW = E // NW
BB = 80
NB = EPW // BB
RPT = NP // NTILE
ZR = 160


def _tc1_body(x_ref, w_ref, alm_ref, arm_ref,
              ha_ref, hb_ref, el_ref, er_ref, mx_ref):
    i = pl.program_id(0)
    h = jnp.dot(x_ref[...], w_ref[...], preferred_element_type=jnp.float32)
    ha_ref[...] = h[:, :HALF]
    hb_ref[...] = h[:, HALF:]
    el = jnp.dot(h, alm_ref[...], preferred_element_type=jnp.float32)
    er = jnp.dot(h, arm_ref[...], preferred_element_type=jnp.float32)
    el_ref[...] = el
    er_ref[...] = er
    blk = jnp.concatenate(
        [jnp.max(el, 0, keepdims=True), jnp.max(er, 0, keepdims=True),
         jnp.full((6, 16), -jnp.inf, jnp.float32)], 0)

    @pl.when(i == 0)
    def _():
        mx_ref[...] = blk

    @pl.when(i > 0)
    def _():
        mx_ref[...] = jnp.maximum(mx_ref[...], blk)


def _prologue(na_ref, nb_ref, dn_ref, rep_ref, b_ref):
    num = jnp.concatenate([na_ref[0] + na_ref[1], nb_ref[0] + nb_ref[1]], 1)
    den = dn_ref[0] + dn_ref[1]
    denb = jnp.dot(den, rep_ref[...], preferred_element_type=jnp.float32)
    return num / (denb + 1e-9) + b_ref[...]


def _tc2_body(na_ref, nb_ref, dn_ref, rep_ref, b_ref, w_ref, alm_ref, arm_ref,
              ha_ref, hb_ref, el_ref, er_ref, mx_ref):
    i = pl.program_id(0)
    a = jnp.maximum(_prologue(na_ref, nb_ref, dn_ref, rep_ref, b_ref), 0.0)
    h = jnp.dot(a, w_ref[...], preferred_element_type=jnp.float32)
    ha_ref[...] = h[:, :HALF]
    hb_ref[...] = h[:, HALF:]
    el = jnp.dot(h, alm_ref[...], preferred_element_type=jnp.float32)
    er = jnp.dot(h, arm_ref[...], preferred_element_type=jnp.float32)
    el_ref[...] = el
    er_ref[...] = er
    blk = jnp.concatenate(
        [jnp.max(el, 0, keepdims=True), jnp.max(er, 0, keepdims=True),
         jnp.full((6, 16), -jnp.inf, jnp.float32)], 0)

    @pl.when(i == 0)
    def _():
        mx_ref[...] = blk

    @pl.when(i > 0)
    def _():
        mx_ref[...] = jnp.maximum(mx_ref[...], blk)


def _ep_body(na_ref, nb_ref, dn_ref, rep_ref, b_ref, o_ref):
    o_ref[...] = _prologue(na_ref, nb_ref, dn_ref, rep_ref, b_ref)


_HSPECS = [pl.BlockSpec((BM, HALF), lambda i: (i, 0)),
           pl.BlockSpec((BM, HALF), lambda i: (i, 0)),
           pl.BlockSpec((BM, 16), lambda i: (i, 0)),
           pl.BlockSpec((BM, 16), lambda i: (i, 0)),
           pl.BlockSpec((8, 16), lambda i: (0, 0))]
_HSHAPES = (jax.ShapeDtypeStruct((NP, HALF), jnp.float32),
            jax.ShapeDtypeStruct((NP, HALF), jnp.float32),
            jax.ShapeDtypeStruct((NP, 16), jnp.float32),
            jax.ShapeDtypeStruct((NP, 16), jnp.float32),
            jax.ShapeDtypeStruct((8, 16), jnp.float32))
_NSPECS = [pl.BlockSpec((NSC, BM, HALF), lambda i: (0, i, 0)),
           pl.BlockSpec((NSC, BM, HALF), lambda i: (0, i, 0)),
           pl.BlockSpec((NSC, BM, 16), lambda i: (0, i, 0)),
           pl.BlockSpec((16, HD), lambda i: (0, 0)),
           pl.BlockSpec((1, HD), lambda i: (0, 0))]


def _tc1(xp, w, alm, arm):
    return pl.pallas_call(
        _tc1_body,
        grid=(NP // BM,),
        in_specs=[pl.BlockSpec((BM, xp.shape[1]), lambda i: (i, 0)),
                  pl.BlockSpec(w.shape, lambda i: (0, 0)),
                  pl.BlockSpec((HD, 16), lambda i: (0, 0)),
                  pl.BlockSpec((HD, 16), lambda i: (0, 0))],
        out_specs=_HSPECS,
        out_shape=_HSHAPES,
    )(xp, w, alm, arm)


def _tc2(na, nb, dn, rep, b2d, w, alm, arm):
    return pl.pallas_call(
        _tc2_body,
        grid=(NP // BM,),
        in_specs=_NSPECS + [pl.BlockSpec((HD, HD), lambda i: (0, 0)),
                            pl.BlockSpec((HD, 16), lambda i: (0, 0)),
                            pl.BlockSpec((HD, 16), lambda i: (0, 0))],
        out_specs=_HSPECS,
        out_shape=_HSHAPES,
    )(na, nb, dn, rep, b2d, w, alm, arm)


def _epilogue(na, nb, dn, rep, b2d):
    return pl.pallas_call(
        _ep_body,
        grid=(NP // BM,),
        in_specs=_NSPECS,
        out_specs=pl.BlockSpec((BM, HD), lambda i: (i, 0)),
        out_shape=jax.ShapeDtypeStruct((NP, HD), jnp.float32),
    )(na, nb, dn, rep, b2d)


def _sweep_body(sweep, do_den,
                h_hbm, el_hbm, er_hbm, src_hbm, dst_hbm, m_hbm, z_hbm, z2_hbm,
                num_hbm, den_hbm,
                num_sh, den_sh, sidx, didx, elr, err, hrows, exb, msgb,
                mbuf):
    c = lax.axis_index("c")
    s = lax.axis_index("s")
    wid = c * NTILE + s
    pltpu.sync_copy(z_hbm, num_sh.at[pl.ds(s * RPT, RPT)])
    if do_den:
        pltpu.sync_copy(z2_hbm, den_sh.at[pl.ds(s * RPT, RPT)])
    pltpu.sync_copy(m_hbm, mbuf)
    plsc.subcore_barrier()
    mvec = mbuf[...]
    base0 = wid * EPW

    def batch(i, carry):
        base = base0 + i * BB
        pltpu.sync_copy(src_hbm.at[pl.ds(base, BB)], sidx)
        pltpu.sync_copy(dst_hbm.at[pl.ds(base, BB)], didx)
        pltpu.sync_copy(el_hbm.at[sidx], elr)
        pltpu.sync_copy(er_hbm.at[didx], err)
        pltpu.sync_copy(h_hbm.at[sidx], hrows)

        def edge(e, cc):
            t = elr[e, :] + err[e, :]
            t = jnp.maximum(t, 0.2 * t)
            ex = jnp.exp(t - mvec)
            exb[e, :] = ex
            for hh in range(4):
                sc_ = ex[4 * sweep + hh]
                for j in range(2):
                    col = hh * DH + j * 16
                    msgb[e, pl.ds(col, 16)] = sc_ * hrows[e, pl.ds(col, 16)]
            return cc

        lax.fori_loop(0, BB, edge, 0)
        pltpu.sync_copy(msgb, num_sh.at[didx], add=True)
        if do_den:
            pltpu.sync_copy(exb, den_sh.at[didx], add=True)
        return carry

    lax.fori_loop(0, NB, batch, 0)
    plsc.subcore_barrier()
    r = s * RPT
    pltpu.sync_copy(num_sh.at[pl.ds(r, RPT)], num_hbm.at[c, pl.ds(r, RPT)])
    if do_den:
        pltpu.sync_copy(den_sh.at[pl.ds(r, RPT)], den_hbm.at[c, pl.ds(r, RPT)])


def _make_sweep(sweep, do_den):
    return pl.kernel(
        functools.partial(_sweep_body, sweep, do_den),
        out_type=(jax.ShapeDtypeStruct((NSC, NP, HALF), jnp.float32),
                  jax.ShapeDtypeStruct((NSC, NP, 16), jnp.float32)),
        mesh=plsc.VectorSubcoreMesh(core_axis_name="c", subcore_axis_name="s"),
        compiler_params=pltpu.CompilerParams(use_tc_tiling_on_sc=False),
        scratch_types=[
            pltpu.VMEM_SHARED((NP, HALF), jnp.float32),
            pltpu.VMEM_SHARED((NP, 16), jnp.float32),
            pltpu.VMEM((BB,), jnp.int32),
            pltpu.VMEM((BB,), jnp.int32),
            pltpu.VMEM((BB, 16), jnp.float32),
            pltpu.VMEM((BB, 16), jnp.float32),
            pltpu.VMEM((BB, HALF), jnp.float32),
            pltpu.VMEM((BB, 16), jnp.float32),
            pltpu.VMEM((BB, HALF), jnp.float32),
            pltpu.VMEM((16,), jnp.float32),
        ],
    )


_sweepA = _make_sweep(0, True)
_sweepB = _make_sweep(1, False)


def _attn_mat(a):
    idx = jnp.arange(HD, dtype=jnp.int32)
    return jnp.zeros((HD, 16), jnp.float32).at[idx, idx // DH].set(a.reshape(-1))


def _mvec(mx):
    return jnp.concatenate([mx[0, :8] + mx[1, :8],
                            jnp.full((8,), 1e30, jnp.float32)])


def kernel(x, edge_index, W1, al1, ar1, b1, W2, al2, ar2, b2):
    src = edge_index[0]
    dst = edge_index[1]
    xp = jnp.pad(x, ((0, NP - N), (0, 0)))
    rep = (jnp.arange(HD)[None, :] // DH ==
           jnp.arange(16)[:, None]).astype(jnp.float32)
    z = jnp.zeros((RPT, HALF), jnp.float32)
    z2 = jnp.zeros((RPT, 16), jnp.float32)

    ha1, hb1, el1, er1, mx1 = _tc1(xp, W1, _attn_mat(al1), _attn_mat(ar1))
    m1 = _mvec(mx1)
    na1, dn1 = _sweepA(ha1, el1, er1, src, dst, m1, z, z2)
    nb1, _ = _sweepB(hb1, el1, er1, src, dst, m1, z, z2)

    ha2, hb2, el2, er2, mx2 = _tc2(na1, nb1, dn1, rep, b1[None, :], W2,
                                   _attn_mat(al2), _attn_mat(ar2))
    m2 = _mvec(mx2)
    na2, dn2 = _sweepA(ha2, el2, er2, src, dst, m2, z, z2)
    nb2, _ = _sweepB(hb2, el2, er2, src, dst, m2, z, z2)

    out = _epilogue(na2, nb2, dn2, rep, b2[None, :])
    return out[:N]


# speedup vs baseline: 29.6678x; 1.5089x over previous
import functools

import jax
import jax.numpy as jnp
from jax import lax
from jax.experimental import pallas as pl
from jax.experimental.pallas import tpu as pltpu
from jax.experimental.pallas import tpu_sc as plsc

N = 10000
E = 320000
HEADS = 8
DH = 32
HD = HEADS * DH
HALF = HD // 2
NP = 10240
BM = 1024
NSC = 2
NTILE = 16
NW = NSC * NTILE
EPW = E // NW
BB = 40
NB = EPW // BB
RPT = NP // NTILE
ZR = 160


def _tc1_body(x_ref, w_ref, alm_ref, arm_ref,
              ha_ref, hb_ref, el_ref, er_ref, mx_ref):
    i = pl.program_id(0)
    h = jnp.dot(x_ref[...], w_ref[...], preferred_element_type=jnp.float32)
    ha_ref[...] = h[:, :HALF]
    hb_ref[...] = h[:, HALF:]
    el = jnp.dot(h, alm_ref[...], preferred_element_type=jnp.float32)
    er = jnp.dot(h, arm_ref[...], preferred_element_type=jnp.float32)
    el_ref[...] = el
    er_ref[...] = er
    blk = jnp.concatenate(
        [jnp.max(el, 0, keepdims=True), jnp.max(er, 0, keepdims=True),
         jnp.full((6, 16), -jnp.inf, jnp.float32)], 0)

    @pl.when(i == 0)
    def _():
        mx_ref[...] = blk

    @pl.when(i > 0)
    def _():
        mx_ref[...] = jnp.maximum(mx_ref[...], blk)


def _prologue(na_ref, nb_ref, dn_ref, rep_ref, b_ref):
    num = jnp.concatenate([na_ref[0] + na_ref[1], nb_ref[0] + nb_ref[1]], 1)
    den = dn_ref[0] + dn_ref[1]
    denb = jnp.dot(den, rep_ref[...], preferred_element_type=jnp.float32)
    return num / (denb + 1e-9) + b_ref[...]


def _tc2_body(na_ref, nb_ref, dn_ref, rep_ref, b_ref, w_ref, alm_ref, arm_ref,
              ha_ref, hb_ref, el_ref, er_ref, mx_ref):
    i = pl.program_id(0)
    a = jnp.maximum(_prologue(na_ref, nb_ref, dn_ref, rep_ref, b_ref), 0.0)
    h = jnp.dot(a, w_ref[...], preferred_element_type=jnp.float32)
    ha_ref[...] = h[:, :HALF]
    hb_ref[...] = h[:, HALF:]
    el = jnp.dot(h, alm_ref[...], preferred_element_type=jnp.float32)
    er = jnp.dot(h, arm_ref[...], preferred_element_type=jnp.float32)
    el_ref[...] = el
    er_ref[...] = er
    blk = jnp.concatenate(
        [jnp.max(el, 0, keepdims=True), jnp.max(er, 0, keepdims=True),
         jnp.full((6, 16), -jnp.inf, jnp.float32)], 0)

    @pl.when(i == 0)
    def _():
        mx_ref[...] = blk

    @pl.when(i > 0)
    def _():
        mx_ref[...] = jnp.maximum(mx_ref[...], blk)


def _ep_body(na_ref, nb_ref, dn_ref, rep_ref, b_ref, o_ref):
    o_ref[...] = _prologue(na_ref, nb_ref, dn_ref, rep_ref, b_ref)


_HSPECS = [pl.BlockSpec((BM, HALF), lambda i: (i, 0)),
           pl.BlockSpec((BM, HALF), lambda i: (i, 0)),
           pl.BlockSpec((BM, 16), lambda i: (i, 0)),
           pl.BlockSpec((BM, 16), lambda i: (i, 0)),
           pl.BlockSpec((8, 16), lambda i: (0, 0))]
_HSHAPES = (jax.ShapeDtypeStruct((NP, HALF), jnp.float32),
            jax.ShapeDtypeStruct((NP, HALF), jnp.float32),
            jax.ShapeDtypeStruct((NP, 16), jnp.float32),
            jax.ShapeDtypeStruct((NP, 16), jnp.float32),
            jax.ShapeDtypeStruct((8, 16), jnp.float32))
_NSPECS = [pl.BlockSpec((NSC, BM, HALF), lambda i: (0, i, 0)),
           pl.BlockSpec((NSC, BM, HALF), lambda i: (0, i, 0)),
           pl.BlockSpec((NSC, BM, 16), lambda i: (0, i, 0)),
           pl.BlockSpec((16, HD), lambda i: (0, 0)),
           pl.BlockSpec((1, HD), lambda i: (0, 0))]


def _tc1(xp, w, alm, arm):
    return pl.pallas_call(
        _tc1_body,
        grid=(NP // BM,),
        in_specs=[pl.BlockSpec((BM, xp.shape[1]), lambda i: (i, 0)),
                  pl.BlockSpec(w.shape, lambda i: (0, 0)),
                  pl.BlockSpec((HD, 16), lambda i: (0, 0)),
                  pl.BlockSpec((HD, 16), lambda i: (0, 0))],
        out_specs=_HSPECS,
        out_shape=_HSHAPES,
    )(xp, w, alm, arm)


def _tc2(na, nb, dn, rep, b2d, w, alm, arm):
    return pl.pallas_call(
        _tc2_body,
        grid=(NP // BM,),
        in_specs=_NSPECS + [pl.BlockSpec((HD, HD), lambda i: (0, 0)),
                            pl.BlockSpec((HD, 16), lambda i: (0, 0)),
                            pl.BlockSpec((HD, 16), lambda i: (0, 0))],
        out_specs=_HSPECS,
        out_shape=_HSHAPES,
    )(na, nb, dn, rep, b2d, w, alm, arm)


def _epilogue(na, nb, dn, rep, b2d):
    return pl.pallas_call(
        _ep_body,
        grid=(NP // BM,),
        in_specs=_NSPECS,
        out_specs=pl.BlockSpec((BM, HD), lambda i: (i, 0)),
        out_shape=jax.ShapeDtypeStruct((NP, HD), jnp.float32),
    )(na, nb, dn, rep, b2d)


def _sweep_body(sweep, do_den,
                h_hbm, el_hbm, er_hbm, src_hbm, dst_hbm, m_hbm, z_hbm, z2_hbm,
                num_hbm, den_hbm,
                num_sh, den_sh, sidx, didx, dscat, elr, err, hrows, exb, msgb,
                mbuf, isem0, isem1, gsem0, gsem1, ssem0, ssem1):
    c = lax.axis_index("c")
    s = lax.axis_index("s")
    wid = c * NTILE + s
    pltpu.sync_copy(z_hbm, num_sh.at[pl.ds(s * RPT, RPT)])
    if do_den:
        pltpu.sync_copy(z2_hbm, den_sh.at[pl.ds(s * RPT, RPT)])
    pltpu.sync_copy(m_hbm, mbuf)
    plsc.subcore_barrier()
    mvec = mbuf[...]
    base0 = wid * EPW
    isem = (isem0, isem1)
    gsem = (gsem0, gsem1)
    ssem = (ssem0, ssem1)

    def idx_copies(i, si):
        base = base0 + i * BB
        return ((src_hbm.at[pl.ds(base, BB)], sidx.at[si], isem[si]),
                (dst_hbm.at[pl.ds(base, BB)], didx.at[si], isem[si]))

    def gather_copies(si):
        return ((el_hbm.at[sidx.at[si]], elr.at[si], gsem[si]),
                (er_hbm.at[didx.at[si]], err.at[si], gsem[si]),
                (h_hbm.at[sidx.at[si]], hrows.at[si], gsem[si]))

    def scatter_copies(si):
        cps = [(msgb.at[si], num_sh.at[dscat.at[si]], ssem[si])]
        if do_den:
            cps.append((exb.at[si], den_sh.at[dscat.at[si]], ssem[si]))
        return cps

    def issue(cps, add=False):
        for src, dst, sem in cps:
            pltpu.async_copy(src, dst, sem, add=add)

    def drain(cps):
        for src, dst, sem in cps:
            pltpu.make_async_copy(src, dst, sem).wait()

    def compute(i, si):
        for off in (0, 16, 24):
            dscat[si, pl.ds(off, 16)] = didx[si, pl.ds(off, 16)]

        def edge(e, cc):
            t = elr[si, e, :] + err[si, e, :]
            t = jnp.maximum(t, 0.2 * t)
            ex = jnp.exp(t - mvec)
            exb[si, e, :] = ex
            for hh in range(4):
                sc_ = ex[4 * sweep + hh]
                for j in range(2):
                    col = hh * DH + j * 16
                    msgb[si, e, pl.ds(col, 16)] = \
                        sc_ * hrows[si, e, pl.ds(col, 16)]
            return cc

        lax.fori_loop(0, BB, edge, 0)

    def step(i, si, do_a, do_c, do_f):
        ni = 1 - si
        if do_a:
            drain(idx_copies(i + 1, ni))
            issue(gather_copies(ni))
        drain(gather_copies(si))
        if do_c:
            drain(scatter_copies(si))
        compute(i, si)
        issue(scatter_copies(si), add=True)
        if do_f:
            issue(idx_copies(i + 2, si))

    issue(idx_copies(0, 0))
    issue(idx_copies(1, 1))
    drain(idx_copies(0, 0))
    issue(gather_copies(0))
    step(0, 0, True, False, True)
    step(1, 1, True, False, True)

    def pair(k, cc):
        i0 = 2 * k
        step(i0, 0, True, True, True)
        step(i0 + 1, 1, True, True, True)
        return cc

    lax.fori_loop(1, NB // 2 - 1, pair, 0)
    step(NB - 2, 0, True, True, False)
    step(NB - 1, 1, False, True, False)
    drain(scatter_copies(0))
    drain(scatter_copies(1))
    plsc.subcore_barrier()
    r = s * RPT
    pltpu.sync_copy(num_sh.at[pl.ds(r, RPT)], num_hbm.at[c, pl.ds(r, RPT)])
    if do_den:
        pltpu.sync_copy(den_sh.at[pl.ds(r, RPT)], den_hbm.at[c, pl.ds(r, RPT)])


def _make_sweep(sweep, do_den):
    return pl.kernel(
        functools.partial(_sweep_body, sweep, do_den),
        out_type=(jax.ShapeDtypeStruct((NSC, NP, HALF), jnp.float32),
                  jax.ShapeDtypeStruct((NSC, NP, 16), jnp.float32)),
        mesh=plsc.VectorSubcoreMesh(core_axis_name="c", subcore_axis_name="s"),
        compiler_params=pltpu.CompilerParams(use_tc_tiling_on_sc=False),
        scratch_types=[
            pltpu.VMEM_SHARED((NP, HALF), jnp.float32),
            pltpu.VMEM_SHARED((NP, 16), jnp.float32),
            pltpu.VMEM((2, BB), jnp.int32),
            pltpu.VMEM((2, BB), jnp.int32),
            pltpu.VMEM((2, BB), jnp.int32),
            pltpu.VMEM((2, BB, 16), jnp.float32),
            pltpu.VMEM((2, BB, 16), jnp.float32),
            pltpu.VMEM((2, BB, HALF), jnp.float32),
            pltpu.VMEM((2, BB, 16), jnp.float32),
            pltpu.VMEM((2, BB, HALF), jnp.float32),
            pltpu.VMEM((16,), jnp.float32),
            pltpu.SemaphoreType.DMA,
            pltpu.SemaphoreType.DMA,
            pltpu.SemaphoreType.DMA,
            pltpu.SemaphoreType.DMA,
            pltpu.SemaphoreType.DMA,
            pltpu.SemaphoreType.DMA,
        ],
    )


_sweepA = _make_sweep(0, True)
_sweepB = _make_sweep(1, False)


def _attn_mat(a):
    idx = jnp.arange(HD, dtype=jnp.int32)
    return jnp.zeros((HD, 16), jnp.float32).at[idx, idx // DH].set(a.reshape(-1))


def _mvec(mx):
    return jnp.concatenate([mx[0, :8] + mx[1, :8],
                            jnp.full((8,), 1e30, jnp.float32)])


def kernel(x, edge_index, W1, al1, ar1, b1, W2, al2, ar2, b2):
    src = edge_index[0]
    dst = edge_index[1]
    xp = jnp.pad(x, ((0, NP - N), (0, 0)))
    rep = (jnp.arange(HD)[None, :] // DH ==
           jnp.arange(16)[:, None]).astype(jnp.float32)
    z = jnp.zeros((RPT, HALF), jnp.float32)
    z2 = jnp.zeros((RPT, 16), jnp.float32)

    ha1, hb1, el1, er1, mx1 = _tc1(xp, W1, _attn_mat(al1), _attn_mat(ar1))
    m1 = _mvec(mx1)
    na1, dn1 = _sweepA(ha1, el1, er1, src, dst, m1, z, z2)
    nb1, _ = _sweepB(hb1, el1, er1, src, dst, m1, z, z2)

    ha2, hb2, el2, er2, mx2 = _tc2(na1, nb1, dn1, rep, b1[None, :], W2,
                                   _attn_mat(al2), _attn_mat(ar2))
    m2 = _mvec(mx2)
    na2, dn2 = _sweepA(ha2, el2, er2, src, dst, m2, z, z2)
    nb2, _ = _sweepB(hb2, el2, er2, src, dst, m2, z, z2)

    out = _epilogue(na2, nb2, dn2, rep, b2[None, :])
    return out[:N]


# speedup vs baseline: 74.8636x; 2.5234x over previous
import functools

import jax
import jax.numpy as jnp
from jax import lax
from jax.experimental import pallas as pl
from jax.experimental.pallas import tpu as pltpu
from jax.experimental.pallas import tpu_sc as plsc

N = 10000
E = 320000
HEADS = 8
DH = 32
HD = HEADS * DH
HALF = HD // 2
NP = 10240
BM = 1024
NSC = 2
NTILE = 16
NW = NSC * NTILE
EPW = E // NW
BB = 40
NB = EPW // BB
RPT = NP // NTILE
ZR = 160


def _tc1_body(x_ref, w_ref, alm_ref, arm_ref,
              ha_ref, hb_ref, el_ref, er_ref, mx_ref):
    i = pl.program_id(0)
    h = jnp.dot(x_ref[...], w_ref[...], preferred_element_type=jnp.float32)
    ha_ref[...] = h[:, :HALF]
    hb_ref[...] = h[:, HALF:]
    el = jnp.dot(h, alm_ref[...], preferred_element_type=jnp.float32)
    er = jnp.dot(h, arm_ref[...], preferred_element_type=jnp.float32)
    el_ref[...] = el
    er_ref[...] = er
    blk = jnp.concatenate(
        [jnp.max(el, 0, keepdims=True), jnp.max(er, 0, keepdims=True),
         jnp.full((6, 16), -jnp.inf, jnp.float32)], 0)

    @pl.when(i == 0)
    def _():
        mx_ref[...] = blk

    @pl.when(i > 0)
    def _():
        mx_ref[...] = jnp.maximum(mx_ref[...], blk)


def _prologue(na_ref, nb_ref, dn_ref, rep_ref, b_ref):
    num = jnp.concatenate([na_ref[0] + na_ref[1], nb_ref[0] + nb_ref[1]], 1)
    den = dn_ref[0] + dn_ref[1]
    denb = jnp.dot(den, rep_ref[...], preferred_element_type=jnp.float32)
    return num / (denb + 1e-9) + b_ref[...]


def _tc2_body(na_ref, nb_ref, dn_ref, rep_ref, b_ref, w_ref, alm_ref, arm_ref,
              ha_ref, hb_ref, el_ref, er_ref, mx_ref):
    i = pl.program_id(0)
    a = jnp.maximum(_prologue(na_ref, nb_ref, dn_ref, rep_ref, b_ref), 0.0)
    h = jnp.dot(a, w_ref[...], preferred_element_type=jnp.float32)
    ha_ref[...] = h[:, :HALF]
    hb_ref[...] = h[:, HALF:]
    el = jnp.dot(h, alm_ref[...], preferred_element_type=jnp.float32)
    er = jnp.dot(h, arm_ref[...], preferred_element_type=jnp.float32)
    el_ref[...] = el
    er_ref[...] = er
    blk = jnp.concatenate(
        [jnp.max(el, 0, keepdims=True), jnp.max(er, 0, keepdims=True),
         jnp.full((6, 16), -jnp.inf, jnp.float32)], 0)

    @pl.when(i == 0)
    def _():
        mx_ref[...] = blk

    @pl.when(i > 0)
    def _():
        mx_ref[...] = jnp.maximum(mx_ref[...], blk)


def _ep_body(na_ref, nb_ref, dn_ref, rep_ref, b_ref, o_ref):
    o_ref[...] = _prologue(na_ref, nb_ref, dn_ref, rep_ref, b_ref)


_HSPECS = [pl.BlockSpec((BM, HALF), lambda i: (i, 0)),
           pl.BlockSpec((BM, HALF), lambda i: (i, 0)),
           pl.BlockSpec((BM, 16), lambda i: (i, 0)),
           pl.BlockSpec((BM, 16), lambda i: (i, 0)),
           pl.BlockSpec((8, 16), lambda i: (0, 0))]
_HSHAPES = (jax.ShapeDtypeStruct((NP, HALF), jnp.float32),
            jax.ShapeDtypeStruct((NP, HALF), jnp.float32),
            jax.ShapeDtypeStruct((NP, 16), jnp.float32),
            jax.ShapeDtypeStruct((NP, 16), jnp.float32),
            jax.ShapeDtypeStruct((8, 16), jnp.float32))
_NSPECS = [pl.BlockSpec((NSC, BM, HALF), lambda i: (0, i, 0)),
           pl.BlockSpec((NSC, BM, HALF), lambda i: (0, i, 0)),
           pl.BlockSpec((NSC, BM, 16), lambda i: (0, i, 0)),
           pl.BlockSpec((16, HD), lambda i: (0, 0)),
           pl.BlockSpec((1, HD), lambda i: (0, 0))]


def _tc1(xp, w, alm, arm):
    return pl.pallas_call(
        _tc1_body,
        grid=(NP // BM,),
        in_specs=[pl.BlockSpec((BM, xp.shape[1]), lambda i: (i, 0)),
                  pl.BlockSpec(w.shape, lambda i: (0, 0)),
                  pl.BlockSpec((HD, 16), lambda i: (0, 0)),
                  pl.BlockSpec((HD, 16), lambda i: (0, 0))],
        out_specs=_HSPECS,
        out_shape=_HSHAPES,
    )(xp, w, alm, arm)


def _tc2(na, nb, dn, rep, b2d, w, alm, arm):
    return pl.pallas_call(
        _tc2_body,
        grid=(NP // BM,),
        in_specs=_NSPECS + [pl.BlockSpec((HD, HD), lambda i: (0, 0)),
                            pl.BlockSpec((HD, 16), lambda i: (0, 0)),
                            pl.BlockSpec((HD, 16), lambda i: (0, 0))],
        out_specs=_HSPECS,
        out_shape=_HSHAPES,
    )(na, nb, dn, rep, b2d, w, alm, arm)


def _epilogue(na, nb, dn, rep, b2d):
    return pl.pallas_call(
        _ep_body,
        grid=(NP // BM,),
        in_specs=_NSPECS,
        out_specs=pl.BlockSpec((BM, HD), lambda i: (i, 0)),
        out_shape=jax.ShapeDtypeStruct((NP, HD), jnp.float32),
    )(na, nb, dn, rep, b2d)


def _sweep_body(sweep, do_den,
                h_hbm, el_hbm, er_hbm, src_hbm, dst_hbm, m_hbm, z_hbm, z2_hbm,
                num_hbm, den_hbm,
                num_sh, den_sh, sidx, didx, dscat, elr, err, hrows, exb, msgb,
                mbuf, isem0, isem1, gsem0, gsem1, ssem0, ssem1):
    c = lax.axis_index("c")
    s = lax.axis_index("s")
    wid = c * NTILE + s
    pltpu.sync_copy(z_hbm, num_sh.at[pl.ds(s * RPT, RPT)])
    if do_den:
        pltpu.sync_copy(z2_hbm, den_sh.at[pl.ds(s * RPT, RPT)])
    pltpu.sync_copy(m_hbm, mbuf)
    plsc.subcore_barrier()
    mvec = mbuf[...]
    base0 = wid * EPW
    isem = (isem0, isem1)
    gsem = (gsem0, gsem1)
    ssem = (ssem0, ssem1)

    def idx_copies(i, si):
        base = base0 + i * BB
        return ((src_hbm.at[pl.ds(base, BB)], sidx.at[si], isem[si]),
                (dst_hbm.at[pl.ds(base, BB)], didx.at[si], isem[si]))

    def gather_copies(si):
        return ((el_hbm.at[sidx.at[si]], elr.at[si], gsem[si]),
                (er_hbm.at[didx.at[si]], err.at[si], gsem[si]),
                (h_hbm.at[sidx.at[si]], hrows.at[si], gsem[si]))

    def scatter_copies(si):
        cps = [(msgb.at[si], num_sh.at[dscat.at[si]], ssem[si])]
        if do_den:
            cps.append((exb.at[si], den_sh.at[dscat.at[si]], ssem[si]))
        return cps

    def issue(cps, add=False):
        for src, dst, sem in cps:
            pltpu.async_copy(src, dst, sem, add=add)

    def drain(cps):
        for src, dst, sem in cps:
            pltpu.make_async_copy(src, dst, sem).wait()

    def compute(i, si):
        for off in (0, 16, 24):
            dscat[si, pl.ds(off, 16)] = didx[si, pl.ds(off, 16)]

        @plsc.parallel_loop(0, BB, unroll=4)
        def _edge(e):
            t = elr[si, e, :] + err[si, e, :]
            t = jnp.maximum(t, 0.2 * t)
            ex = jnp.exp(t - mvec)
            exb[si, e, :] = ex
            for hh in range(4):
                sc_ = ex[4 * sweep + hh]
                for j in range(2):
                    col = hh * DH + j * 16
                    msgb[si, e, pl.ds(col, 16)] = \
                        sc_ * hrows[si, e, pl.ds(col, 16)]

    def step(i, si, do_a, do_c, do_f):
        ni = 1 - si
        if do_a:
            drain(idx_copies(i + 1, ni))
            issue(gather_copies(ni))
        drain(gather_copies(si))
        if do_c:
            drain(scatter_copies(si))
        compute(i, si)
        issue(scatter_copies(si), add=True)
        if do_f:
            issue(idx_copies(i + 2, si))

    issue(idx_copies(0, 0))
    issue(idx_copies(1, 1))
    drain(idx_copies(0, 0))
    issue(gather_copies(0))
    step(0, 0, True, False, True)
    step(1, 1, True, False, True)

    def pair(k, cc):
        i0 = 2 * k
        step(i0, 0, True, True, True)
        step(i0 + 1, 1, True, True, True)
        return cc

    lax.fori_loop(1, NB // 2 - 1, pair, 0)
    step(NB - 2, 0, True, True, False)
    step(NB - 1, 1, False, True, False)
    drain(scatter_copies(0))
    drain(scatter_copies(1))
    plsc.subcore_barrier()
    r = s * RPT
    pltpu.sync_copy(num_sh.at[pl.ds(r, RPT)], num_hbm.at[c, pl.ds(r, RPT)])
    if do_den:
        pltpu.sync_copy(den_sh.at[pl.ds(r, RPT)], den_hbm.at[c, pl.ds(r, RPT)])


def _make_sweep(sweep, do_den):
    return pl.kernel(
        functools.partial(_sweep_body, sweep, do_den),
        out_type=(jax.ShapeDtypeStruct((NSC, NP, HALF), jnp.float32),
                  jax.ShapeDtypeStruct((NSC, NP, 16), jnp.float32)),
        mesh=plsc.VectorSubcoreMesh(core_axis_name="c", subcore_axis_name="s"),
        compiler_params=pltpu.CompilerParams(use_tc_tiling_on_sc=False),
        scratch_types=[
            pltpu.VMEM_SHARED((NP, HALF), jnp.float32),
            pltpu.VMEM_SHARED((NP, 16), jnp.float32),
            pltpu.VMEM((2, BB), jnp.int32),
            pltpu.VMEM((2, BB), jnp.int32),
            pltpu.VMEM((2, BB), jnp.int32),
            pltpu.VMEM((2, BB, 16), jnp.float32),
            pltpu.VMEM((2, BB, 16), jnp.float32),
            pltpu.VMEM((2, BB, HALF), jnp.float32),
            pltpu.VMEM((2, BB, 16), jnp.float32),
            pltpu.VMEM((2, BB, HALF), jnp.float32),
            pltpu.VMEM((16,), jnp.float32),
            pltpu.SemaphoreType.DMA,
            pltpu.SemaphoreType.DMA,
            pltpu.SemaphoreType.DMA,
            pltpu.SemaphoreType.DMA,
            pltpu.SemaphoreType.DMA,
            pltpu.SemaphoreType.DMA,
        ],
    )


_sweepA = _make_sweep(0, True)
_sweepB = _make_sweep(1, False)


def _attn_mat(a):
    idx = jnp.arange(HD, dtype=jnp.int32)
    return jnp.zeros((HD, 16), jnp.float32).at[idx, idx // DH].set(a.reshape(-1))


def _mvec(mx):
    return jnp.concatenate([mx[0, :8] + mx[1, :8],
                            jnp.full((8,), 1e30, jnp.float32)])


def kernel(x, edge_index, W1, al1, ar1, b1, W2, al2, ar2, b2):
    src = edge_index[0]
    dst = edge_index[1]
    xp = jnp.pad(x, ((0, NP - N), (0, 0)))
    rep = (jnp.arange(HD)[None, :] // DH ==
           jnp.arange(16)[:, None]).astype(jnp.float32)
    z = jnp.zeros((RPT, HALF), jnp.float32)
    z2 = jnp.zeros((RPT, 16), jnp.float32)

    ha1, hb1, el1, er1, mx1 = _tc1(xp, W1, _attn_mat(al1), _attn_mat(ar1))
    m1 = _mvec(mx1)
    na1, dn1 = _sweepA(ha1, el1, er1, src, dst, m1, z, z2)
    nb1, _ = _sweepB(hb1, el1, er1, src, dst, m1, z, z2)

    ha2, hb2, el2, er2, mx2 = _tc2(na1, nb1, dn1, rep, b1[None, :], W2,
                                   _attn_mat(al2), _attn_mat(ar2))
    m2 = _mvec(mx2)
    na2, dn2 = _sweepA(ha2, el2, er2, src, dst, m2, z, z2)
    nb2, _ = _sweepB(hb2, el2, er2, src, dst, m2, z, z2)

    out = _epilogue(na2, nb2, dn2, rep, b2[None, :])
    return out[:N]
